# baseline (speedup 1.0000x reference)
import functools

import numpy as np
import jax
import jax.numpy as jnp
from jax import lax
from jax.experimental import pallas as pl
from jax.experimental.pallas import tpu as pltpu
from jax.experimental.pallas import tpu_sc as plsc

TOP_K = 512
SK = 4096
SKP = SK + 64
NROWS = 8192
NW = 32
ROWS_PER_W = NROWS // NW
L = 16
CAP = 576
MIN_I32 = -(2**31)

_S_NEG1E9 = int(np.float32(-1e9).view(np.int32))
KS_NEG1E9 = _S_NEG1E9 ^ 0x7FFFFFFF
KU_NEG1E9 = KS_NEG1E9 ^ MIN_I32


def _lanes():
  return lax.broadcasted_iota(jnp.int32, (L,), 0)


def _scalar_at(vec, lane):
  lanes = _lanes()
  return jnp.sum(jnp.where(lanes == lane, vec, 0))


def _find_bucket(hist_ref, target):
  lanes = _lanes()
  tv = plsc.load_gather(hist_ref, [lanes * L])
  for c in range(1, L):
    tv = tv + plsc.load_gather(hist_ref, [lanes * L + c])
  above = lax.rev(plsc.cumsum(lax.rev(tv, (0,))), (0,))
  excl = above - tv
  mstar = jnp.max(jnp.where(above >= target, lanes, -1))
  exm = _scalar_at(excl, mstar)
  h = hist_ref[pl.ds(mstar << 4, L)]
  rc = lax.rev(plsc.cumsum(lax.rev(h, (0,))), (0,))
  cum_ge = exm + rc
  c_in = jnp.max(jnp.where(cum_ge >= target, lanes, -1))
  c = (mstar << 4) | c_in
  n_ge = _scalar_at(cum_ge, c_in)
  hc = _scalar_at(h, c_in)
  return c, n_ge - hc, n_ge


def _zero(ref, nwords):
  z = jnp.zeros((L,), jnp.int32)

  def body(i, _):
    for u in range(4):
      ref[pl.ds((i * 4 + u) * L, L)] = z
    return 0

  lax.fori_loop(0, nwords // (4 * L), body, 0)


def _radix_pass(src_k, src_v, dst_k, dst_v, hist_v, off_v, exb, shift,
                dedup_hist):
  lanes = _lanes()
  ones = jnp.ones((L,), jnp.int32)
  _zero(hist_v, 256)

  def hist_body(i, _):
    for u in range(4):
      k = src_k[pl.ds((i * 4 + u) * L, L)]
      d = lax.shift_right_logical(k, shift) & 255
      if dedup_hist:
        occ, lastm = plsc.scan_count(d)
        plsc.addupdate_scatter(hist_v, [d], occ, mask=lastm)
      else:
        plsc.addupdate_scatter(hist_v, [d], ones)
    return 0

  lax.fori_loop(0, CAP // (4 * L), hist_body, 0)

  tv = plsc.load_gather(hist_v, [lanes * L])
  for c in range(1, L):
    tv = tv + plsc.load_gather(hist_v, [lanes * L + c])
  above = lax.rev(plsc.cumsum(lax.rev(tv, (0,))), (0,))
  exb[pl.ds(0, L)] = above - tv

  def off_body(m, _):
    h = hist_v[pl.ds(m * L, L)]
    rc = lax.rev(plsc.cumsum(lax.rev(h, (0,))), (0,))
    carry = plsc.load_gather(exb, [jnp.full((L,), m, jnp.int32)])
    off_v[pl.ds(m * L, L)] = carry + rc - h
    return 0

  lax.fori_loop(0, 16, off_body, 0)

  def perm_body(i, _):
    for u in range(4):
      k = src_k[pl.ds((i * 4 + u) * L, L)]
      v = src_v[pl.ds((i * 4 + u) * L, L)]
      d = lax.shift_right_logical(k, shift) & 255
      occ, lastm = plsc.scan_count(d)
      base = plsc.load_gather(off_v, [d])
      dest = base + occ - 1
      plsc.store_scatter(dst_k, [dest], k)
      plsc.store_scatter(dst_v, [dest], v)
      plsc.addupdate_scatter(off_v, [d], occ, mask=lastm)
    return 0

  lax.fori_loop(0, CAP // (4 * L), perm_body, 0)


def _body(scores_hbm, idx_hbm, maskw_hbm, row_a, row_b, key_v, hist_v, off_v,
          exb, cka, cva, ckb, cvb, cvout_a, cvout_b, maskw_a, maskw_b, sc0,
          sem_in_a, sem_in_b, sem_out_a, sem_out_b):
  wid = lax.axis_index("s") * 2 + lax.axis_index("c")
  lanes = _lanes()
  ones = jnp.ones((L,), jnp.int32)

  def one_row(it, rbuf, sem_in, rbuf_next, sem_in_next, cvout, maskw_v,
              sem_out):
    row = it * NW + wid
    q = row & (SK - 1)
    nv = (q + L) >> 4

    with jax.named_scope("sc_wait_in"):
      pltpu.make_async_copy(scores_hbm.at[row, pl.ds(0, SK)], rbuf.at[pl.ds(0, SK)], sem_in).wait()

    @pl.when(it + 1 < ROWS_PER_W)
    def _():
      pltpu.async_copy(scores_hbm.at[row + NW, pl.ds(0, SK)],
                       rbuf_next.at[pl.ds(0, SK)], sem_in_next)

    @pl.when(it >= 2)
    def _():
      rp = row - 2 * NW
      pltpu.make_async_copy(cvout.at[pl.ds(0, TOP_K)], idx_hbm.at[rp],
                            sem_out).wait()
      pltpu.make_async_copy(maskw_v, maskw_hbm.at[rp], sem_out).wait()

    sc_p1 = jax.named_scope("sc_p1"); sc_p1.__enter__()
    _zero(hist_v, 256)

    def p1_body(i, _):
      for u in range(4):
        iv = i * 4 + u
        s = rbuf[pl.ds(iv * L, L)]
        ks = jnp.where(s < 0, s ^ 0x7FFFFFFF, s)
        valid = (iv * L + lanes) <= q
        ks = jnp.where(valid, ks, MIN_I32)
        key_v[pl.ds(iv * L, L)] = ks
        d1 = lax.shift_right_logical(ks ^ MIN_I32, 24)
        plsc.addupdate_scatter(hist_v, [d1], ones, mask=valid)
      return 0

    lax.fori_loop(0, (nv + 3) >> 2, p1_body, 0)
    sc_p1.__exit__(None, None, None)
    sc_sel = jax.named_scope("sc_select"); sc_sel.__enter__()

    sc0[0] = MIN_I32
    sc0[1] = q + 1

    @pl.when(q >= TOP_K - 1)
    def _():
      c1, n_gt1, _n_ge1 = _find_bucket(hist_v, jnp.int32(TOP_K))
      _zero(hist_v, 256)

      def p2_body(i, _):
        for u in range(4):
          iv = i * 4 + u
          ks = key_v[pl.ds(iv * L, L)]
          ku = ks ^ MIN_I32
          m2 = (lax.shift_right_logical(ku, 24) == c1) & (ks != MIN_I32)
          d2 = lax.shift_right_logical(ku, 16) & 255
          plsc.addupdate_scatter(hist_v, [d2], ones, mask=m2)
        return 0

      lax.fori_loop(0, (nv + 3) >> 2, p2_body, 0)
      c2, _n_gt2, n_ge2 = _find_bucket(hist_v, TOP_K - n_gt1)
      t_u = (c1 << 24) | (c2 << 16)
      sc0[0] = t_u ^ MIN_I32
      sc0[1] = n_gt1 + n_ge2

    sc_sel.__exit__(None, None, None)
    sc_cp = jax.named_scope("sc_compact"); sc_cp.__enter__()
    t_s = jnp.maximum(sc0[0], MIN_I32 + 1)
    n_cand = sc0[1]

    def fill_body(i, _):
      cka[pl.ds(i * L, L)] = jnp.full((L,), KU_NEG1E9, jnp.int32)
      cva[pl.ds(i * L, L)] = i * L + lanes
      return 0

    lax.fori_loop(n_cand >> 4, CAP // L, fill_body, 0)

    def compact_body(i, off):
      for u in range(4):
        iv = i * 4 + u
        ks = key_v[pl.ds(iv * L, L)]
        m = ks >= t_s
        pos = plsc.cumsum(ones, mask=m)
        dest = off + pos - 1
        dm = m & (dest < CAP)
        plsc.store_scatter(cka, [dest], ks ^ MIN_I32, mask=dm)
        plsc.store_scatter(cva, [dest], iv * L + lanes, mask=dm)
        off = off + plsc.all_reduce_population_count(m)
      return off

    lax.fori_loop(0, (nv + 3) >> 2, compact_body, jnp.zeros((L,), jnp.int32))
    sc_cp.__exit__(None, None, None)

    sc_st = jax.named_scope("sc_sort"); sc_st.__enter__()
    _radix_pass(cka, cva, ckb, cvb, hist_v, off_v, exb, 0, False)
    _radix_pass(ckb, cvb, cka, cva, hist_v, off_v, exb, 8, False)
    _radix_pass(cka, cva, ckb, cvb, hist_v, off_v, exb, 16, True)
    _radix_pass(ckb, cvb, cka, cvout, hist_v, off_v, exb, 24, True)
    sc_st.__exit__(None, None, None)
    sc_mk = jax.named_scope("sc_mask"); sc_mk.__enter__()

    _zero(maskw_v, SK // 4)

    def mask_body(i, _):
      for u in range(2):
        j = cvout[pl.ds((i * 2 + u) * L, L)]
        w = lax.shift_right_logical(j, 2)
        val = jnp.left_shift(ones, (j & 3) << 3)
        for t in range(4):
          plsc.addupdate_scatter(maskw_v, [w], val, mask=(j & 3) == t)
      return 0

    lax.fori_loop(0, TOP_K // (2 * L), mask_body, 0)

    sc_mk.__exit__(None, None, None)
    pltpu.async_copy(cvout.at[pl.ds(0, TOP_K)], idx_hbm.at[row], sem_out)
    pltpu.async_copy(maskw_v, maskw_hbm.at[row], sem_out)

  pltpu.async_copy(scores_hbm.at[wid, pl.ds(0, SK)], row_a.at[pl.ds(0, SK)],
                   sem_in_a)

  def pair_body(i2, _):
    one_row(2 * i2, row_a, sem_in_a, row_b, sem_in_b, cvout_a, maskw_a,
            sem_out_a)
    one_row(2 * i2 + 1, row_b, sem_in_b, row_a, sem_in_a, cvout_b, maskw_b,
            sem_out_b)
    return 0

  lax.fori_loop(0, ROWS_PER_W // 2, pair_body, 0)

  for p, (cvout, maskw_v, sem_out) in enumerate(
      ((cvout_a, maskw_a, sem_out_a), (cvout_b, maskw_b, sem_out_b))):
    rlast = (ROWS_PER_W - 2 + p) * NW + wid
    pltpu.make_async_copy(cvout.at[pl.ds(0, TOP_K)], idx_hbm.at[rlast],
                          sem_out).wait()
    pltpu.make_async_copy(maskw_v, maskw_hbm.at[rlast], sem_out).wait()


@functools.partial(jax.jit, static_argnums=())
def _topk_sc(scores):
  mesh = plsc.VectorSubcoreMesh(core_axis_name="c", subcore_axis_name="s")
  f = pl.kernel(
      _body,
      out_type=(
          jax.ShapeDtypeStruct((NROWS, TOP_K), jnp.int32),
          jax.ShapeDtypeStruct((NROWS, SK // 4), jnp.int32),
      ),
      mesh=mesh,
      compiler_params=pltpu.CompilerParams(needs_layout_passes=False),
      scratch_types=[
          pltpu.VMEM((SKP,), jnp.int32),
          pltpu.VMEM((SKP,), jnp.int32),
          pltpu.VMEM((SKP,), jnp.int32),
          pltpu.VMEM((256,), jnp.int32),
          pltpu.VMEM((256,), jnp.int32),
          pltpu.VMEM((L,), jnp.int32),
          pltpu.VMEM((CAP,), jnp.int32),
          pltpu.VMEM((CAP,), jnp.int32),
          pltpu.VMEM((CAP,), jnp.int32),
          pltpu.VMEM((CAP,), jnp.int32),
          pltpu.VMEM((CAP,), jnp.int32),
          pltpu.VMEM((CAP,), jnp.int32),
          pltpu.VMEM((SK // 4,), jnp.int32),
          pltpu.VMEM((SK // 4,), jnp.int32),
          pltpu.SMEM((8,), jnp.int32),
          pltpu.SemaphoreType.DMA,
          pltpu.SemaphoreType.DMA,
          pltpu.SemaphoreType.DMA,
          pltpu.SemaphoreType.DMA,
      ],
  )
  return f(scores)


def kernel(index_scores):
  B, Sq, Sk = index_scores.shape
  scores = lax.bitcast_convert_type(index_scores, jnp.int32).reshape(B * Sq, Sk)
  idx_out, maskw = _topk_sc(scores)
  top_k_indices = idx_out.reshape(B, Sq, TOP_K)
  mask_bytes = lax.bitcast_convert_type(maskw, jnp.uint8)
  top_k_mask = mask_bytes.reshape(B, Sq, Sk) != 0
  sparsity = jnp.float32(1.0) - jnp.float32(TOP_K) / jnp.float32(Sk)
  return (top_k_mask, top_k_indices, sparsity)


# speedup vs baseline: 1.0299x; 1.0299x over previous
import functools

import numpy as np
import jax
import jax.numpy as jnp
from jax import lax
from jax.experimental import pallas as pl
from jax.experimental.pallas import tpu as pltpu
from jax.experimental.pallas import tpu_sc as plsc

TOP_K = 512
SK = 4096
SKP = SK + 64
NROWS = 8192
NW = 32
ROWS_PER_W = NROWS // NW
L = 16
CAP = 576
MIN_I32 = -(2**31)

_S_NEG1E9 = int(np.float32(-1e9).view(np.int32))
KS_NEG1E9 = _S_NEG1E9 ^ 0x7FFFFFFF
KU_NEG1E9 = KS_NEG1E9 ^ MIN_I32


def _lanes():
  return lax.broadcasted_iota(jnp.int32, (L,), 0)


def _scalar_at(vec, lane):
  lanes = _lanes()
  return jnp.sum(jnp.where(lanes == lane, vec, 0))


def _find_bucket(hist_ref, target):
  lanes = _lanes()
  tv = plsc.load_gather(hist_ref, [lanes * L])
  for c in range(1, L):
    tv = tv + plsc.load_gather(hist_ref, [lanes * L + c])
  above = lax.rev(plsc.cumsum(lax.rev(tv, (0,))), (0,))
  excl = above - tv
  mstar = jnp.max(jnp.where(above >= target, lanes, -1))
  exm = _scalar_at(excl, mstar)
  h = hist_ref[pl.ds(mstar << 4, L)]
  rc = lax.rev(plsc.cumsum(lax.rev(h, (0,))), (0,))
  cum_ge = exm + rc
  c_in = jnp.max(jnp.where(cum_ge >= target, lanes, -1))
  c = (mstar << 4) | c_in
  n_ge = _scalar_at(cum_ge, c_in)
  hc = _scalar_at(h, c_in)
  return c, n_ge - hc, n_ge


def _zero(ref, nwords):
  z = jnp.zeros((L,), jnp.int32)

  def body(i, _):
    for u in range(4):
      ref[pl.ds((i * 4 + u) * L, L)] = z
    return 0

  lax.fori_loop(0, nwords // (4 * L), body, 0)


def _radix_pass(src_k, src_v, dst_k, dst_v, hist_v, off_v, exb, shift,
                dedup_hist):
  lanes = _lanes()
  ones = jnp.ones((L,), jnp.int32)
  _zero(hist_v, 256)

  def hist_body(i, _):
    for u in range(4):
      k = src_k[pl.ds((i * 4 + u) * L, L)]
      d = lax.shift_right_logical(k, shift) & 255
      if dedup_hist:
        occ, lastm = plsc.scan_count(d)
        plsc.addupdate_scatter(hist_v, [d], occ, mask=lastm)
      else:
        plsc.addupdate_scatter(hist_v, [d], ones)
    return 0

  lax.fori_loop(0, CAP // (4 * L), hist_body, 0)

  tv = plsc.load_gather(hist_v, [lanes * L])
  for c in range(1, L):
    tv = tv + plsc.load_gather(hist_v, [lanes * L + c])
  above = lax.rev(plsc.cumsum(lax.rev(tv, (0,))), (0,))
  exb[pl.ds(0, L)] = above - tv

  def off_body(m, _):
    h = hist_v[pl.ds(m * L, L)]
    rc = lax.rev(plsc.cumsum(lax.rev(h, (0,))), (0,))
    carry = plsc.load_gather(exb, [jnp.full((L,), m, jnp.int32)])
    off_v[pl.ds(m * L, L)] = carry + rc - h
    return 0

  lax.fori_loop(0, 16, off_body, 0)

  def perm_body(i, _):
    for u in range(4):
      k = src_k[pl.ds((i * 4 + u) * L, L)]
      v = src_v[pl.ds((i * 4 + u) * L, L)]
      d = lax.shift_right_logical(k, shift) & 255
      occ, lastm = plsc.scan_count(d)
      base = plsc.load_gather(off_v, [d])
      dest = base + occ - 1
      plsc.store_scatter(dst_k, [dest], k)
      plsc.store_scatter(dst_v, [dest], v)
      plsc.addupdate_scatter(off_v, [d], occ, mask=lastm)
    return 0

  lax.fori_loop(0, CAP // (4 * L), perm_body, 0)


def _body(scores_hbm, idx_hbm, maskw_hbm, row_a, row_b, key_v, hist_v, off_v,
          exb, cka, cva, ckb, cvb, cvout_a, cvout_b, maskw_a, maskw_b, sc0,
          sem_in_a, sem_in_b, sem_out_a, sem_out_b):
  wid = lax.axis_index("s") * 2 + lax.axis_index("c")
  lanes = _lanes()
  ones = jnp.ones((L,), jnp.int32)

  def one_row(it, rbuf, sem_in, rbuf_next, sem_in_next, cvout, maskw_v,
              sem_out):
    row = it * NW + wid
    q = row & (SK - 1)
    nv = (q + L) >> 4

    with jax.named_scope("sc_wait_in"):
      pltpu.make_async_copy(scores_hbm.at[row, pl.ds(0, SK)], rbuf.at[pl.ds(0, SK)], sem_in).wait()

    @pl.when(it + 1 < ROWS_PER_W)
    def _():
      pltpu.async_copy(scores_hbm.at[row + NW, pl.ds(0, SK)],
                       rbuf_next.at[pl.ds(0, SK)], sem_in_next)

    @pl.when(it >= 2)
    def _():
      rp = row - 2 * NW
      pltpu.make_async_copy(cvout.at[pl.ds(0, TOP_K)], idx_hbm.at[rp],
                            sem_out).wait()
      pltpu.make_async_copy(maskw_v, maskw_hbm.at[rp], sem_out).wait()

    sc_p1 = jax.named_scope("sc_p1"); sc_p1.__enter__()
    _zero(hist_v, 256)

    def p1_body(i, _):
      for u in range(4):
        iv = i * 4 + u
        s = rbuf[pl.ds(iv * L, L)]
        ks = jnp.where(s < 0, s ^ 0x7FFFFFFF, s)
        valid = (iv * L + lanes) <= q
        ks = jnp.where(valid, ks, MIN_I32)
        key_v[pl.ds(iv * L, L)] = ks
        d1 = lax.shift_right_logical(ks ^ MIN_I32, 24)
        plsc.addupdate_scatter(hist_v, [d1], ones, mask=valid)
      return 0

    lax.fori_loop(0, (nv + 3) >> 2, p1_body, 0)
    sc_p1.__exit__(None, None, None)
    sc_sel = jax.named_scope("sc_select"); sc_sel.__enter__()

    sc0[0] = MIN_I32
    sc0[1] = q + 1

    @pl.when(q >= TOP_K - 1)
    def _():
      c1, n_gt1, _n_ge1 = _find_bucket(hist_v, jnp.int32(TOP_K))
      _zero(hist_v, 256)

      def p2_body(i, _):
        for u in range(4):
          iv = i * 4 + u
          ks = key_v[pl.ds(iv * L, L)]
          ku = ks ^ MIN_I32
          m2 = (lax.shift_right_logical(ku, 24) == c1) & (ks != MIN_I32)
          d2 = lax.shift_right_logical(ku, 16) & 255
          plsc.addupdate_scatter(hist_v, [d2], ones, mask=m2)
        return 0

      lax.fori_loop(0, (nv + 3) >> 2, p2_body, 0)
      c2, _n_gt2, n_ge2 = _find_bucket(hist_v, TOP_K - n_gt1)
      t_u = (c1 << 24) | (c2 << 16)
      sc0[0] = t_u ^ MIN_I32
      sc0[1] = n_gt1 + n_ge2

    sc_sel.__exit__(None, None, None)
    sc_cp = jax.named_scope("sc_compact"); sc_cp.__enter__()
    t_s = jnp.maximum(sc0[0], MIN_I32 + 1)
    n_cand = sc0[1]

    def fill_body(i, _):
      cka[pl.ds(i * L, L)] = jnp.full((L,), KU_NEG1E9, jnp.int32)
      cva[pl.ds(i * L, L)] = i * L + lanes
      return 0

    lax.fori_loop(n_cand >> 4, CAP // L, fill_body, 0)

    def compact_body(i, off):
      for u in range(4):
        iv = i * 4 + u
        ks = key_v[pl.ds(iv * L, L)]
        m = ks >= t_s
        pos = plsc.cumsum(ones, mask=m)
        dest = off + pos - 1
        dm = m & (dest < CAP)
        plsc.store_scatter(cka, [dest], ks ^ MIN_I32, mask=dm)
        plsc.store_scatter(cva, [dest], iv * L + lanes, mask=dm)
        off = off + plsc.all_reduce_population_count(m)
      return off

    lax.fori_loop(0, (nv + 3) >> 2, compact_body, jnp.zeros((L,), jnp.int32))
    sc_cp.__exit__(None, None, None)

    sc_st = jax.named_scope("sc_sort"); sc_st.__enter__()
    _radix_pass(cka, cva, ckb, cvb, hist_v, off_v, exb, 0, False)
    _radix_pass(ckb, cvb, cka, cva, hist_v, off_v, exb, 8, False)
    _radix_pass(cka, cva, ckb, cvb, hist_v, off_v, exb, 16, False)
    _radix_pass(ckb, cvb, cka, cvout, hist_v, off_v, exb, 24, False)
    sc_st.__exit__(None, None, None)
    sc_mk = jax.named_scope("sc_mask"); sc_mk.__enter__()

    _zero(maskw_v, SK // 4)

    def mask_body(i, _):
      for u in range(2):
        j = cvout[pl.ds((i * 2 + u) * L, L)]
        w = lax.shift_right_logical(j, 2)
        val = jnp.left_shift(ones, (j & 3) << 3)
        for t in range(4):
          plsc.addupdate_scatter(maskw_v, [w], val, mask=(j & 3) == t)
      return 0

    lax.fori_loop(0, TOP_K // (2 * L), mask_body, 0)

    sc_mk.__exit__(None, None, None)
    pltpu.async_copy(cvout.at[pl.ds(0, TOP_K)], idx_hbm.at[row], sem_out)
    pltpu.async_copy(maskw_v, maskw_hbm.at[row], sem_out)

  pltpu.async_copy(scores_hbm.at[wid, pl.ds(0, SK)], row_a.at[pl.ds(0, SK)],
                   sem_in_a)

  def pair_body(i2, _):
    one_row(2 * i2, row_a, sem_in_a, row_b, sem_in_b, cvout_a, maskw_a,
            sem_out_a)
    one_row(2 * i2 + 1, row_b, sem_in_b, row_a, sem_in_a, cvout_b, maskw_b,
            sem_out_b)
    return 0

  lax.fori_loop(0, ROWS_PER_W // 2, pair_body, 0)

  for p, (cvout, maskw_v, sem_out) in enumerate(
      ((cvout_a, maskw_a, sem_out_a), (cvout_b, maskw_b, sem_out_b))):
    rlast = (ROWS_PER_W - 2 + p) * NW + wid
    pltpu.make_async_copy(cvout.at[pl.ds(0, TOP_K)], idx_hbm.at[rlast],
                          sem_out).wait()
    pltpu.make_async_copy(maskw_v, maskw_hbm.at[rlast], sem_out).wait()


@functools.partial(jax.jit, static_argnums=())
def _topk_sc(scores):
  mesh = plsc.VectorSubcoreMesh(core_axis_name="c", subcore_axis_name="s")
  f = pl.kernel(
      _body,
      out_type=(
          jax.ShapeDtypeStruct((NROWS, TOP_K), jnp.int32),
          jax.ShapeDtypeStruct((NROWS, SK // 4), jnp.int32),
      ),
      mesh=mesh,
      compiler_params=pltpu.CompilerParams(needs_layout_passes=False),
      scratch_types=[
          pltpu.VMEM((SKP,), jnp.int32),
          pltpu.VMEM((SKP,), jnp.int32),
          pltpu.VMEM((SKP,), jnp.int32),
          pltpu.VMEM((256,), jnp.int32),
          pltpu.VMEM((256,), jnp.int32),
          pltpu.VMEM((L,), jnp.int32),
          pltpu.VMEM((CAP,), jnp.int32),
          pltpu.VMEM((CAP,), jnp.int32),
          pltpu.VMEM((CAP,), jnp.int32),
          pltpu.VMEM((CAP,), jnp.int32),
          pltpu.VMEM((CAP,), jnp.int32),
          pltpu.VMEM((CAP,), jnp.int32),
          pltpu.VMEM((SK // 4,), jnp.int32),
          pltpu.VMEM((SK // 4,), jnp.int32),
          pltpu.SMEM((8,), jnp.int32),
          pltpu.SemaphoreType.DMA,
          pltpu.SemaphoreType.DMA,
          pltpu.SemaphoreType.DMA,
          pltpu.SemaphoreType.DMA,
      ],
  )
  return f(scores)


def kernel(index_scores):
  B, Sq, Sk = index_scores.shape
  scores = lax.bitcast_convert_type(index_scores, jnp.int32).reshape(B * Sq, Sk)
  idx_out, maskw = _topk_sc(scores)
  top_k_indices = idx_out.reshape(B, Sq, TOP_K)
  mask_bytes = lax.bitcast_convert_type(maskw, jnp.uint8)
  top_k_mask = mask_bytes.reshape(B, Sq, Sk) != 0
  sparsity = jnp.float32(1.0) - jnp.float32(TOP_K) / jnp.float32(Sk)
  return (top_k_mask, top_k_indices, sparsity)


# speedup vs baseline: 1.0348x; 1.0047x over previous
import functools

import numpy as np
import jax
import jax.numpy as jnp
from jax import lax
from jax.experimental import pallas as pl
from jax.experimental.pallas import tpu as pltpu
from jax.experimental.pallas import tpu_sc as plsc

TOP_K = 512
SK = 4096
SKP = SK + 64
NROWS = 8192
NW = 32
NPAIR = SK // NW
L = 16
CAP = 576
MIN_I32 = -(2**31)

_S_NEG1E9 = int(np.float32(-1e9).view(np.int32))
KS_NEG1E9 = _S_NEG1E9 ^ 0x7FFFFFFF
KU_NEG1E9 = KS_NEG1E9 ^ MIN_I32


def _lanes():
  return lax.broadcasted_iota(jnp.int32, (L,), 0)


def _scalar_at(vec, lane):
  lanes = _lanes()
  return jnp.sum(jnp.where(lanes == lane, vec, 0))


def _find_bucket(hist_ref, target):
  lanes = _lanes()
  tv = plsc.load_gather(hist_ref, [lanes * L])
  for c in range(1, L):
    tv = tv + plsc.load_gather(hist_ref, [lanes * L + c])
  above = lax.rev(plsc.cumsum(lax.rev(tv, (0,))), (0,))
  excl = above - tv
  mstar = jnp.max(jnp.where(above >= target, lanes, -1))
  exm = _scalar_at(excl, mstar)
  h = hist_ref[pl.ds(mstar << 4, L)]
  rc = lax.rev(plsc.cumsum(lax.rev(h, (0,))), (0,))
  cum_ge = exm + rc
  c_in = jnp.max(jnp.where(cum_ge >= target, lanes, -1))
  c = (mstar << 4) | c_in
  n_ge = _scalar_at(cum_ge, c_in)
  hc = _scalar_at(h, c_in)
  return c, n_ge - hc, n_ge


def _zero2(ref0, ref1, nwords):
  z = jnp.zeros((L,), jnp.int32)

  def body(i, _):
    for u in range(2):
      ref0[pl.ds((i * 2 + u) * L, L)] = z
      ref1[pl.ds((i * 2 + u) * L, L)] = z
    return 0

  lax.fori_loop(0, nwords // (2 * L), body, 0)


def _radix_pass2(b0, b1, shift):
  lanes = _lanes()
  ones = jnp.ones((L,), jnp.int32)
  _zero2(b0[4], b1[4], 256)

  def hist_body(i, _):
    for h in (b0, b1):
      for u in range(2):
        k = h[0][pl.ds((i * 2 + u) * L, L)]
        d = lax.shift_right_logical(k, shift) & 255
        plsc.addupdate_scatter(h[4], [d], ones)
    return 0

  lax.fori_loop(0, CAP // (2 * L), hist_body, 0)

  for h in (b0, b1):
    tv = plsc.load_gather(h[4], [lanes * L])
    for c in range(1, L):
      tv = tv + plsc.load_gather(h[4], [lanes * L + c])
    above = lax.rev(plsc.cumsum(lax.rev(tv, (0,))), (0,))
    h[6][pl.ds(0, L)] = above - tv

  def off_body(m, _):
    mi = jnp.full((L,), m, jnp.int32)
    for h in (b0, b1):
      hh = h[4][pl.ds(m * L, L)]
      rc = lax.rev(plsc.cumsum(lax.rev(hh, (0,))), (0,))
      carry = plsc.load_gather(h[6], [mi])
      h[5][pl.ds(m * L, L)] = carry + rc - hh
    return 0

  lax.fori_loop(0, 16, off_body, 0)

  def perm_body(i, _):
    for h in (b0, b1):
      for u in range(2):
        k = h[0][pl.ds((i * 2 + u) * L, L)]
        v = h[1][pl.ds((i * 2 + u) * L, L)]
        d = lax.shift_right_logical(k, shift) & 255
        occ, lastm = plsc.scan_count(d)
        base = plsc.load_gather(h[5], [d])
        dest = base + occ - 1
        plsc.store_scatter(h[2], [dest], k)
        plsc.store_scatter(h[3], [dest], v)
        plsc.addupdate_scatter(h[5], [d], occ, mask=lastm)
    return 0

  lax.fori_loop(0, CAP // (2 * L), perm_body, 0)


def _body(scores_hbm, idx_hbm, maskw_hbm,
          rin_a0, rin_a1, rin_b0, rin_b1,
          key0, key1, hist0, hist1, off0, off1, exb0, exb1,
          cka0, cva0, ckb0, cvb0, cka1, cva1, ckb1, cvb1,
          cvout_a0, cvout_a1, cvout_b0, cvout_b1,
          maskw_a0, maskw_a1, maskw_b0, maskw_b1, sc0,
          sem_in_a, sem_in_b, sem_out_a, sem_out_b):
  wid = lax.axis_index("s") * 2 + lax.axis_index("c")
  lanes = _lanes()
  ones = jnp.ones((L,), jnp.int32)

  def one_pair(it, rin0, rin1, sem_in, rin_n0, rin_n1, sem_in_next,
               cvout0, cvout1, maskw0, maskw1, sem_out):
    q = it * NW + wid
    row1 = q + SK
    nv = (q + L) >> 4
    ntrip = (nv + 3) >> 2

    with jax.named_scope("sc_wait_in"):
      pltpu.make_async_copy(scores_hbm.at[q, pl.ds(0, SK)],
                            rin0.at[pl.ds(0, SK)], sem_in).wait()
      pltpu.make_async_copy(scores_hbm.at[row1, pl.ds(0, SK)],
                            rin1.at[pl.ds(0, SK)], sem_in).wait()

    @pl.when(it + 1 < NPAIR)
    def _():
      pltpu.async_copy(scores_hbm.at[q + NW, pl.ds(0, SK)],
                       rin_n0.at[pl.ds(0, SK)], sem_in_next)
      pltpu.async_copy(scores_hbm.at[row1 + NW, pl.ds(0, SK)],
                       rin_n1.at[pl.ds(0, SK)], sem_in_next)

    @pl.when(it >= 2)
    def _():
      qp = q - 2 * NW
      pltpu.make_async_copy(cvout0.at[pl.ds(0, TOP_K)], idx_hbm.at[qp],
                            sem_out).wait()
      pltpu.make_async_copy(maskw0, maskw_hbm.at[qp], sem_out).wait()
      pltpu.make_async_copy(cvout1.at[pl.ds(0, TOP_K)], idx_hbm.at[qp + SK],
                            sem_out).wait()
      pltpu.make_async_copy(maskw1, maskw_hbm.at[qp + SK], sem_out).wait()

    sc_p1 = jax.named_scope("sc_p1"); sc_p1.__enter__()
    _zero2(hist0, hist1, 256)

    def p1_body(i, _):
      for rbuf, key_v, hist_v in ((rin0, key0, hist0), (rin1, key1, hist1)):
        for u in range(2):
          iv = i * 2 + u
          s = rbuf[pl.ds(iv * L, L)]
          ks = jnp.where(s < 0, s ^ 0x7FFFFFFF, s)
          valid = (iv * L + lanes) <= q
          ks = jnp.where(valid, ks, MIN_I32)
          key_v[pl.ds(iv * L, L)] = ks
          d1 = lax.shift_right_logical(ks ^ MIN_I32, 24)
          plsc.addupdate_scatter(hist_v, [d1], ones, mask=valid)
      return 0

    lax.fori_loop(0, (nv + 1) >> 1, p1_body, 0)
    sc_p1.__exit__(None, None, None)
    sc_sel = jax.named_scope("sc_select"); sc_sel.__enter__()

    sc0[0] = MIN_I32
    sc0[1] = q + 1
    sc0[2] = MIN_I32
    sc0[3] = q + 1

    @pl.when(q >= TOP_K - 1)
    def _():
      c1a, n_gt1a, _ = _find_bucket(hist0, jnp.int32(TOP_K))
      c1b, n_gt1b, _ = _find_bucket(hist1, jnp.int32(TOP_K))
      _zero2(hist0, hist1, 256)

      def p2_body(i, _):
        for key_v, hist_v, c1 in ((key0, hist0, c1a), (key1, hist1, c1b)):
          for u in range(2):
            iv = i * 2 + u
            ks = key_v[pl.ds(iv * L, L)]
            ku = ks ^ MIN_I32
            m2 = (lax.shift_right_logical(ku, 24) == c1) & (ks != MIN_I32)
            d2 = lax.shift_right_logical(ku, 16) & 255
            plsc.addupdate_scatter(hist_v, [d2], ones, mask=m2)
        return 0

      lax.fori_loop(0, (nv + 1) >> 1, p2_body, 0)
      c2a, _g, n_ge2a = _find_bucket(hist0, TOP_K - n_gt1a)
      c2b, _g, n_ge2b = _find_bucket(hist1, TOP_K - n_gt1b)
      sc0[0] = ((c1a << 24) | (c2a << 16)) ^ MIN_I32
      sc0[1] = n_gt1a + n_ge2a
      sc0[2] = ((c1b << 24) | (c2b << 16)) ^ MIN_I32
      sc0[3] = n_gt1b + n_ge2b

    sc_sel.__exit__(None, None, None)
    sc_cp = jax.named_scope("sc_compact"); sc_cp.__enter__()
    ts0 = jnp.maximum(sc0[0], MIN_I32 + 1)
    ts1 = jnp.maximum(sc0[2], MIN_I32 + 1)

    def make_fill(cka, cva):
      def fill_body(i, _):
        cka[pl.ds(i * L, L)] = jnp.full((L,), KU_NEG1E9, jnp.int32)
        cva[pl.ds(i * L, L)] = i * L + lanes
        return 0
      return fill_body

    lax.fori_loop(sc0[1] >> 4, CAP // L, make_fill(cka0, cva0), 0)
    lax.fori_loop(sc0[3] >> 4, CAP // L, make_fill(cka1, cva1), 0)

    def compact_body(i, offs):
      off0, off1 = offs
      for u in range(2):
        iv = i * 2 + u
        ks = key0[pl.ds(iv * L, L)]
        m = ks >= ts0
        pos = plsc.cumsum(ones, mask=m)
        dest = off0 + pos - 1
        dm = m & (dest < CAP)
        plsc.store_scatter(cka0, [dest], ks ^ MIN_I32, mask=dm)
        plsc.store_scatter(cva0, [dest], iv * L + lanes, mask=dm)
        off0 = off0 + plsc.all_reduce_population_count(m)
        ks = key1[pl.ds(iv * L, L)]
        m = ks >= ts1
        pos = plsc.cumsum(ones, mask=m)
        dest = off1 + pos - 1
        dm = m & (dest < CAP)
        plsc.store_scatter(cka1, [dest], ks ^ MIN_I32, mask=dm)
        plsc.store_scatter(cva1, [dest], iv * L + lanes, mask=dm)
        off1 = off1 + plsc.all_reduce_population_count(m)
      return off0, off1

    z16 = jnp.zeros((L,), jnp.int32)
    lax.fori_loop(0, (nv + 1) >> 1, compact_body, (z16, z16))
    sc_cp.__exit__(None, None, None)

    sc_st = jax.named_scope("sc_sort"); sc_st.__enter__()
    s0 = (cka0, cva0, ckb0, cvb0, hist0, off0, exb0)
    r0 = (ckb0, cvb0, cka0, cva0, hist0, off0, exb0)
    s1 = (cka1, cva1, ckb1, cvb1, hist1, off1, exb1)
    r1 = (ckb1, cvb1, cka1, cva1, hist1, off1, exb1)
    fin0 = (ckb0, cvb0, cka0, cvout0, hist0, off0, exb0)
    fin1 = (ckb1, cvb1, cka1, cvout1, hist1, off1, exb1)
    _radix_pass2(s0, s1, 0)
    _radix_pass2(r0, r1, 8)
    _radix_pass2(s0, s1, 16)
    _radix_pass2(fin0, fin1, 24)
    sc_st.__exit__(None, None, None)
    sc_mk = jax.named_scope("sc_mask"); sc_mk.__enter__()

    _zero2(maskw0, maskw1, SK // 4)

    def mask_body(i, _):
      for cvout, maskw_v in ((cvout0, maskw0), (cvout1, maskw1)):
        j = cvout[pl.ds(i * L, L)]
        w = lax.shift_right_logical(j, 2)
        val = jnp.left_shift(ones, (j & 3) << 3)
        for t in range(4):
          plsc.addupdate_scatter(maskw_v, [w], val, mask=(j & 3) == t)
      return 0

    lax.fori_loop(0, TOP_K // L, mask_body, 0)

    sc_mk.__exit__(None, None, None)
    pltpu.async_copy(cvout0.at[pl.ds(0, TOP_K)], idx_hbm.at[q], sem_out)
    pltpu.async_copy(maskw0, maskw_hbm.at[q], sem_out)
    pltpu.async_copy(cvout1.at[pl.ds(0, TOP_K)], idx_hbm.at[row1], sem_out)
    pltpu.async_copy(maskw1, maskw_hbm.at[row1], sem_out)

  pltpu.async_copy(scores_hbm.at[wid, pl.ds(0, SK)], rin_a0.at[pl.ds(0, SK)],
                   sem_in_a)
  pltpu.async_copy(scores_hbm.at[wid + SK, pl.ds(0, SK)],
                   rin_a1.at[pl.ds(0, SK)], sem_in_a)

  def pair_body(i2, _):
    one_pair(2 * i2, rin_a0, rin_a1, sem_in_a, rin_b0, rin_b1, sem_in_b,
             cvout_a0, cvout_a1, maskw_a0, maskw_a1, sem_out_a)
    one_pair(2 * i2 + 1, rin_b0, rin_b1, sem_in_b, rin_a0, rin_a1, sem_in_a,
             cvout_b0, cvout_b1, maskw_b0, maskw_b1, sem_out_b)
    return 0

  lax.fori_loop(0, NPAIR // 2, pair_body, 0)

  for p, (cv0, cv1, mk0, mk1, sem_out) in enumerate(
      ((cvout_a0, cvout_a1, maskw_a0, maskw_a1, sem_out_a),
       (cvout_b0, cvout_b1, maskw_b0, maskw_b1, sem_out_b))):
    qlast = (NPAIR - 2 + p) * NW + wid
    pltpu.make_async_copy(cv0.at[pl.ds(0, TOP_K)], idx_hbm.at[qlast],
                          sem_out).wait()
    pltpu.make_async_copy(mk0, maskw_hbm.at[qlast], sem_out).wait()
    pltpu.make_async_copy(cv1.at[pl.ds(0, TOP_K)], idx_hbm.at[qlast + SK],
                          sem_out).wait()
    pltpu.make_async_copy(mk1, maskw_hbm.at[qlast + SK], sem_out).wait()


@functools.partial(jax.jit, static_argnums=())
def _topk_sc(scores):
  mesh = plsc.VectorSubcoreMesh(core_axis_name="c", subcore_axis_name="s")
  vm = lambda n: pltpu.VMEM((n,), jnp.int32)
  f = pl.kernel(
      _body,
      out_type=(
          jax.ShapeDtypeStruct((NROWS, TOP_K), jnp.int32),
          jax.ShapeDtypeStruct((NROWS, SK // 4), jnp.int32),
      ),
      mesh=mesh,
      compiler_params=pltpu.CompilerParams(needs_layout_passes=False),
      scratch_types=[
          vm(SKP), vm(SKP), vm(SKP), vm(SKP),
          vm(SKP), vm(SKP),
          vm(256), vm(256), vm(256), vm(256), vm(L), vm(L),
          vm(CAP), vm(CAP), vm(CAP), vm(CAP),
          vm(CAP), vm(CAP), vm(CAP), vm(CAP),
          vm(CAP), vm(CAP), vm(CAP), vm(CAP),
          vm(SK // 4), vm(SK // 4), vm(SK // 4), vm(SK // 4),
          pltpu.SMEM((8,), jnp.int32),
          pltpu.SemaphoreType.DMA,
          pltpu.SemaphoreType.DMA,
          pltpu.SemaphoreType.DMA,
          pltpu.SemaphoreType.DMA,
      ],
  )
  return f(scores)


def kernel(index_scores):
  B, Sq, Sk = index_scores.shape
  scores = lax.bitcast_convert_type(index_scores, jnp.int32).reshape(B * Sq, Sk)
  idx_out, maskw = _topk_sc(scores)
  top_k_indices = idx_out.reshape(B, Sq, TOP_K)
  mask_bytes = lax.bitcast_convert_type(maskw, jnp.uint8)
  top_k_mask = mask_bytes.reshape(B, Sq, Sk) != 0
  sparsity = jnp.float32(1.0) - jnp.float32(TOP_K) / jnp.float32(Sk)
  return (top_k_mask, top_k_indices, sparsity)


# speedup vs baseline: 1.4439x; 1.3954x over previous
import functools

import numpy as np
import jax
import jax.numpy as jnp
from jax import lax
from jax.experimental import pallas as pl
from jax.experimental.pallas import tpu as pltpu
from jax.experimental.pallas import tpu_sc as plsc

TOP_K = 512
SK = 4096
SKP = SK + 64
NROWS = 8192
NW = 32
NPAIR = SK // NW
L = 16
CAP = 576
MIN_I32 = -(2**31)

_S_NEG1E9 = int(np.float32(-1e9).view(np.int32))
KS_NEG1E9 = _S_NEG1E9 ^ 0x7FFFFFFF
KU_NEG1E9 = KS_NEG1E9 ^ MIN_I32


def _lanes():
  return lax.broadcasted_iota(jnp.int32, (L,), 0)


def _scalar_at(vec, lane):
  lanes = _lanes()
  return jnp.sum(jnp.where(lanes == lane, vec, 0))


def _find_bucket(hist_ref, target):
  lanes = _lanes()
  tv = plsc.load_gather(hist_ref, [lanes * L])
  for c in range(1, L):
    tv = tv + plsc.load_gather(hist_ref, [lanes * L + c])
  above = lax.rev(plsc.cumsum(lax.rev(tv, (0,))), (0,))
  excl = above - tv
  mstar = jnp.max(jnp.where(above >= target, lanes, -1))
  exm = _scalar_at(excl, mstar)
  h = hist_ref[pl.ds(mstar << 4, L)]
  rc = lax.rev(plsc.cumsum(lax.rev(h, (0,))), (0,))
  cum_ge = exm + rc
  c_in = jnp.max(jnp.where(cum_ge >= target, lanes, -1))
  c = (mstar << 4) | c_in
  n_ge = _scalar_at(cum_ge, c_in)
  hc = _scalar_at(h, c_in)
  return c, n_ge - hc, n_ge


def _zero2(ref0, ref1, nwords):
  z = jnp.zeros((L,), jnp.int32)

  @plsc.parallel_loop(0, nwords // (2 * L), unroll=2)
  def _(i):
    for u in range(2):
      ref0[pl.ds((i * 2 + u) * L, L)] = z
      ref1[pl.ds((i * 2 + u) * L, L)] = z


def _radix_pass2(b0, b1, shift):
  lanes = _lanes()
  ones = jnp.ones((L,), jnp.int32)
  _zero2(b0[4], b1[4], 256)

  @plsc.parallel_loop(0, CAP // (2 * L), unroll=2)
  def _(i):
    for h in (b0, b1):
      for u in range(2):
        k = h[0][pl.ds((i * 2 + u) * L, L)]
        d = lax.shift_right_logical(k, shift) & 255
        plsc.addupdate_scatter(h[4], [d], ones)

  for h in (b0, b1):
    tv = plsc.load_gather(h[4], [lanes * L])
    for c in range(1, L):
      tv = tv + plsc.load_gather(h[4], [lanes * L + c])
    above = lax.rev(plsc.cumsum(lax.rev(tv, (0,))), (0,))
    h[6][pl.ds(0, L)] = above - tv

  @plsc.parallel_loop(0, 16, unroll=2)
  def _(m):
    mi = jnp.full((L,), m, jnp.int32)
    for h in (b0, b1):
      hh = h[4][pl.ds(m * L, L)]
      rc = lax.rev(plsc.cumsum(lax.rev(hh, (0,))), (0,))
      carry = plsc.load_gather(h[6], [mi])
      h[5][pl.ds(m * L, L)] = carry + rc - hh

  def perm_body(i, _):
    for h in (b0, b1):
      for u in range(2):
        k = h[0][pl.ds((i * 2 + u) * L, L)]
        v = h[1][pl.ds((i * 2 + u) * L, L)]
        d = lax.shift_right_logical(k, shift) & 255
        occ, lastm = plsc.scan_count(d)
        base = plsc.load_gather(h[5], [d])
        dest = base + occ - 1
        plsc.store_scatter(h[2], [dest], k)
        plsc.store_scatter(h[3], [dest], v)
        plsc.addupdate_scatter(h[5], [d], occ, mask=lastm)
    return 0

  lax.fori_loop(0, CAP // (2 * L), perm_body, 0)


def _body(scores_hbm, idx_hbm, maskw_hbm,
          rin_a0, rin_a1, rin_b0, rin_b1,
          key0, key1, hist0, hist1, off0, off1, exb0, exb1,
          cka0, cva0, ckb0, cvb0, cka1, cva1, ckb1, cvb1,
          cvout_a0, cvout_a1, cvout_b0, cvout_b1,
          maskw_a0, maskw_a1, maskw_b0, maskw_b1, sc0,
          sem_in_a, sem_in_b, sem_out_a, sem_out_b):
  wid = lax.axis_index("s") * 2 + lax.axis_index("c")
  lanes = _lanes()
  ones = jnp.ones((L,), jnp.int32)

  def one_pair(it, rin0, rin1, sem_in, rin_n0, rin_n1, sem_in_next,
               cvout0, cvout1, maskw0, maskw1, sem_out):
    q = it * NW + wid
    row1 = q + SK
    nv = (q + L) >> 4
    ntrip = (nv + 3) >> 2

    with jax.named_scope("sc_wait_in"):
      pltpu.make_async_copy(scores_hbm.at[q, pl.ds(0, SK)],
                            rin0.at[pl.ds(0, SK)], sem_in).wait()
      pltpu.make_async_copy(scores_hbm.at[row1, pl.ds(0, SK)],
                            rin1.at[pl.ds(0, SK)], sem_in).wait()

    @pl.when(it + 1 < NPAIR)
    def _():
      pltpu.async_copy(scores_hbm.at[q + NW, pl.ds(0, SK)],
                       rin_n0.at[pl.ds(0, SK)], sem_in_next)
      pltpu.async_copy(scores_hbm.at[row1 + NW, pl.ds(0, SK)],
                       rin_n1.at[pl.ds(0, SK)], sem_in_next)

    @pl.when(it >= 2)
    def _():
      qp = q - 2 * NW
      pltpu.make_async_copy(cvout0.at[pl.ds(0, TOP_K)], idx_hbm.at[qp],
                            sem_out).wait()
      pltpu.make_async_copy(maskw0, maskw_hbm.at[qp], sem_out).wait()
      pltpu.make_async_copy(cvout1.at[pl.ds(0, TOP_K)], idx_hbm.at[qp + SK],
                            sem_out).wait()
      pltpu.make_async_copy(maskw1, maskw_hbm.at[qp + SK], sem_out).wait()

    sc_p1 = jax.named_scope("sc_p1"); sc_p1.__enter__()
    _zero2(hist0, hist1, 256)

    @plsc.parallel_loop(0, (nv + 1) >> 1, unroll=2)
    def _(i):
      for rbuf, key_v, hist_v in ((rin0, key0, hist0), (rin1, key1, hist1)):
        for u in range(2):
          iv = i * 2 + u
          s = rbuf[pl.ds(iv * L, L)]
          ks = jnp.where(s < 0, s ^ 0x7FFFFFFF, s)
          valid = (iv * L + lanes) <= q
          ks = jnp.where(valid, ks, MIN_I32)
          key_v[pl.ds(iv * L, L)] = ks
          d1 = lax.shift_right_logical(ks ^ MIN_I32, 24)
          plsc.addupdate_scatter(hist_v, [d1], ones, mask=valid)
    sc_p1.__exit__(None, None, None)
    sc_sel = jax.named_scope("sc_select"); sc_sel.__enter__()

    sc0[0] = MIN_I32
    sc0[1] = q + 1
    sc0[2] = MIN_I32
    sc0[3] = q + 1

    @pl.when(q >= TOP_K - 1)
    def _():
      c1a, n_gt1a, _ = _find_bucket(hist0, jnp.int32(TOP_K))
      c1b, n_gt1b, _ = _find_bucket(hist1, jnp.int32(TOP_K))
      _zero2(hist0, hist1, 256)

      @plsc.parallel_loop(0, (nv + 1) >> 1, unroll=2)
      def _(i):
        for key_v, hist_v, c1 in ((key0, hist0, c1a), (key1, hist1, c1b)):
          for u in range(2):
            iv = i * 2 + u
            ks = key_v[pl.ds(iv * L, L)]
            ku = ks ^ MIN_I32
            m2 = (lax.shift_right_logical(ku, 24) == c1) & (ks != MIN_I32)
            d2 = lax.shift_right_logical(ku, 16) & 255
            plsc.addupdate_scatter(hist_v, [d2], ones, mask=m2)
      c2a, _g, n_ge2a = _find_bucket(hist0, TOP_K - n_gt1a)
      c2b, _g, n_ge2b = _find_bucket(hist1, TOP_K - n_gt1b)
      sc0[0] = ((c1a << 24) | (c2a << 16)) ^ MIN_I32
      sc0[1] = n_gt1a + n_ge2a
      sc0[2] = ((c1b << 24) | (c2b << 16)) ^ MIN_I32
      sc0[3] = n_gt1b + n_ge2b

    sc_sel.__exit__(None, None, None)
    sc_cp = jax.named_scope("sc_compact"); sc_cp.__enter__()
    ts0 = jnp.maximum(sc0[0], MIN_I32 + 1)
    ts1 = jnp.maximum(sc0[2], MIN_I32 + 1)

    for cka, cva, slot in ((cka0, cva0, 1), (cka1, cva1, 3)):
      @plsc.parallel_loop(sc0[slot] >> 4, CAP // L, unroll=2)
      def _(i):
        cka[pl.ds(i * L, L)] = jnp.full((L,), KU_NEG1E9, jnp.int32)
        cva[pl.ds(i * L, L)] = i * L + lanes

    def compact_body(i, offs):
      off0, off1 = offs
      for u in range(2):
        iv = i * 2 + u
        ks = key0[pl.ds(iv * L, L)]
        m = ks >= ts0
        pos = plsc.cumsum(ones, mask=m)
        dest = off0 + pos - 1
        dm = m & (dest < CAP)
        plsc.store_scatter(cka0, [dest], ks ^ MIN_I32, mask=dm)
        plsc.store_scatter(cva0, [dest], iv * L + lanes, mask=dm)
        off0 = off0 + plsc.all_reduce_population_count(m)
        ks = key1[pl.ds(iv * L, L)]
        m = ks >= ts1
        pos = plsc.cumsum(ones, mask=m)
        dest = off1 + pos - 1
        dm = m & (dest < CAP)
        plsc.store_scatter(cka1, [dest], ks ^ MIN_I32, mask=dm)
        plsc.store_scatter(cva1, [dest], iv * L + lanes, mask=dm)
        off1 = off1 + plsc.all_reduce_population_count(m)
      return off0, off1

    z16 = jnp.zeros((L,), jnp.int32)
    lax.fori_loop(0, (nv + 1) >> 1, compact_body, (z16, z16))
    sc_cp.__exit__(None, None, None)

    sc_st = jax.named_scope("sc_sort"); sc_st.__enter__()
    s0 = (cka0, cva0, ckb0, cvb0, hist0, off0, exb0)
    r0 = (ckb0, cvb0, cka0, cva0, hist0, off0, exb0)
    s1 = (cka1, cva1, ckb1, cvb1, hist1, off1, exb1)
    r1 = (ckb1, cvb1, cka1, cva1, hist1, off1, exb1)
    fin0 = (ckb0, cvb0, cka0, cvout0, hist0, off0, exb0)
    fin1 = (ckb1, cvb1, cka1, cvout1, hist1, off1, exb1)
    _radix_pass2(s0, s1, 0)
    _radix_pass2(r0, r1, 8)
    _radix_pass2(s0, s1, 16)
    _radix_pass2(fin0, fin1, 24)
    sc_st.__exit__(None, None, None)
    sc_mk = jax.named_scope("sc_mask"); sc_mk.__enter__()

    _zero2(maskw0, maskw1, SK // 4)

    @plsc.parallel_loop(0, TOP_K // L, unroll=2)
    def _(i):
      for cvout, maskw_v in ((cvout0, maskw0), (cvout1, maskw1)):
        j = cvout[pl.ds(i * L, L)]
        w = lax.shift_right_logical(j, 2)
        val = jnp.left_shift(ones, (j & 3) << 3)
        for t in range(4):
          plsc.addupdate_scatter(maskw_v, [w], val, mask=(j & 3) == t)

    sc_mk.__exit__(None, None, None)
    pltpu.async_copy(cvout0.at[pl.ds(0, TOP_K)], idx_hbm.at[q], sem_out)
    pltpu.async_copy(maskw0, maskw_hbm.at[q], sem_out)
    pltpu.async_copy(cvout1.at[pl.ds(0, TOP_K)], idx_hbm.at[row1], sem_out)
    pltpu.async_copy(maskw1, maskw_hbm.at[row1], sem_out)

  pltpu.async_copy(scores_hbm.at[wid, pl.ds(0, SK)], rin_a0.at[pl.ds(0, SK)],
                   sem_in_a)
  pltpu.async_copy(scores_hbm.at[wid + SK, pl.ds(0, SK)],
                   rin_a1.at[pl.ds(0, SK)], sem_in_a)

  def pair_body(i2, _):
    one_pair(2 * i2, rin_a0, rin_a1, sem_in_a, rin_b0, rin_b1, sem_in_b,
             cvout_a0, cvout_a1, maskw_a0, maskw_a1, sem_out_a)
    one_pair(2 * i2 + 1, rin_b0, rin_b1, sem_in_b, rin_a0, rin_a1, sem_in_a,
             cvout_b0, cvout_b1, maskw_b0, maskw_b1, sem_out_b)
    return 0

  lax.fori_loop(0, NPAIR // 2, pair_body, 0)

  for p, (cv0, cv1, mk0, mk1, sem_out) in enumerate(
      ((cvout_a0, cvout_a1, maskw_a0, maskw_a1, sem_out_a),
       (cvout_b0, cvout_b1, maskw_b0, maskw_b1, sem_out_b))):
    qlast = (NPAIR - 2 + p) * NW + wid
    pltpu.make_async_copy(cv0.at[pl.ds(0, TOP_K)], idx_hbm.at[qlast],
                          sem_out).wait()
    pltpu.make_async_copy(mk0, maskw_hbm.at[qlast], sem_out).wait()
    pltpu.make_async_copy(cv1.at[pl.ds(0, TOP_K)], idx_hbm.at[qlast + SK],
                          sem_out).wait()
    pltpu.make_async_copy(mk1, maskw_hbm.at[qlast + SK], sem_out).wait()


@functools.partial(jax.jit, static_argnums=())
def _topk_sc(scores):
  mesh = plsc.VectorSubcoreMesh(core_axis_name="c", subcore_axis_name="s")
  vm = lambda n: pltpu.VMEM((n,), jnp.int32)
  f = pl.kernel(
      _body,
      out_type=(
          jax.ShapeDtypeStruct((NROWS, TOP_K), jnp.int32),
          jax.ShapeDtypeStruct((NROWS, SK // 4), jnp.int32),
      ),
      mesh=mesh,
      compiler_params=pltpu.CompilerParams(needs_layout_passes=False),
      scratch_types=[
          vm(SKP), vm(SKP), vm(SKP), vm(SKP),
          vm(SKP), vm(SKP),
          vm(256), vm(256), vm(256), vm(256), vm(L), vm(L),
          vm(CAP), vm(CAP), vm(CAP), vm(CAP),
          vm(CAP), vm(CAP), vm(CAP), vm(CAP),
          vm(CAP), vm(CAP), vm(CAP), vm(CAP),
          vm(SK // 4), vm(SK // 4), vm(SK // 4), vm(SK // 4),
          pltpu.SMEM((8,), jnp.int32),
          pltpu.SemaphoreType.DMA,
          pltpu.SemaphoreType.DMA,
          pltpu.SemaphoreType.DMA,
          pltpu.SemaphoreType.DMA,
      ],
  )
  return f(scores)


def kernel(index_scores):
  B, Sq, Sk = index_scores.shape
  scores = lax.bitcast_convert_type(index_scores, jnp.int32).reshape(B * Sq, Sk)
  idx_out, maskw = _topk_sc(scores)
  top_k_indices = idx_out.reshape(B, Sq, TOP_K)
  mask_bytes = lax.bitcast_convert_type(maskw, jnp.uint8)
  top_k_mask = mask_bytes.reshape(B, Sq, Sk) != 0
  sparsity = jnp.float32(1.0) - jnp.float32(TOP_K) / jnp.float32(Sk)
  return (top_k_mask, top_k_indices, sparsity)


# speedup vs baseline: 1.6166x; 1.1196x over previous
import functools

import numpy as np
import jax
import jax.numpy as jnp
from jax import lax
from jax.experimental import pallas as pl
from jax.experimental.pallas import tpu as pltpu
from jax.experimental.pallas import tpu_sc as plsc

TOP_K = 512
SK = 4096
SKP = SK + 64
NROWS = 8192
NW = 32
NPAIR = SK // NW
L = 16
CAP = 576
MIN_I32 = -(2**31)

_S_NEG1E9 = int(np.float32(-1e9).view(np.int32))
KS_NEG1E9 = _S_NEG1E9 ^ 0x7FFFFFFF
KU_NEG1E9 = KS_NEG1E9 ^ MIN_I32


def _lanes():
  return lax.broadcasted_iota(jnp.int32, (L,), 0)


def _scalar_at(vec, lane):
  lanes = _lanes()
  return jnp.sum(jnp.where(lanes == lane, vec, 0))


def _find_bucket(hist_ref, target):
  lanes = _lanes()
  tv = plsc.load_gather(hist_ref, [lanes * L])
  for c in range(1, L):
    tv = tv + plsc.load_gather(hist_ref, [lanes * L + c])
  above = lax.rev(plsc.cumsum(lax.rev(tv, (0,))), (0,))
  excl = above - tv
  mstar = jnp.max(jnp.where(above >= target, lanes, -1))
  exm = _scalar_at(excl, mstar)
  h = hist_ref[pl.ds(mstar << 4, L)]
  rc = lax.rev(plsc.cumsum(lax.rev(h, (0,))), (0,))
  cum_ge = exm + rc
  c_in = jnp.max(jnp.where(cum_ge >= target, lanes, -1))
  c = (mstar << 4) | c_in
  n_ge = _scalar_at(cum_ge, c_in)
  hc = _scalar_at(h, c_in)
  return c, n_ge - hc, n_ge


def _zero2(ref0, ref1, nwords):
  z = jnp.zeros((L,), jnp.int32)

  @plsc.parallel_loop(0, nwords // (2 * L), unroll=2)
  def _(i):
    for u in range(2):
      ref0[pl.ds((i * 2 + u) * L, L)] = z
      ref1[pl.ds((i * 2 + u) * L, L)] = z


def _radix_pass2(b0, b1, shift, ntrip2):
  lanes = _lanes()
  ones = jnp.ones((L,), jnp.int32)
  _zero2(b0[4], b1[4], 256)

  @plsc.parallel_loop(0, ntrip2, unroll=2)
  def _(i):
    for h in (b0, b1):
      for u in range(2):
        k = h[0][pl.ds((i * 2 + u) * L, L)]
        d = lax.shift_right_logical(k, shift) & 255
        plsc.addupdate_scatter(h[4], [d], ones)

  for h in (b0, b1):
    tv = plsc.load_gather(h[4], [lanes * L])
    for c in range(1, L):
      tv = tv + plsc.load_gather(h[4], [lanes * L + c])
    above = lax.rev(plsc.cumsum(lax.rev(tv, (0,))), (0,))
    h[6][pl.ds(0, L)] = above - tv

  @plsc.parallel_loop(0, 16, unroll=2)
  def _(m):
    mi = jnp.full((L,), m, jnp.int32)
    for h in (b0, b1):
      hh = h[4][pl.ds(m * L, L)]
      rc = lax.rev(plsc.cumsum(lax.rev(hh, (0,))), (0,))
      carry = plsc.load_gather(h[6], [mi])
      h[5][pl.ds(m * L, L)] = carry + rc - hh

  def perm_body(i, _):
    for h in (b0, b1):
      for u in range(2):
        k = h[0][pl.ds((i * 2 + u) * L, L)]
        v = h[1][pl.ds((i * 2 + u) * L, L)]
        d = lax.shift_right_logical(k, shift) & 255
        occ, lastm = plsc.scan_count(d)
        base = plsc.load_gather(h[5], [d])
        dest = base + occ - 1
        plsc.store_scatter(h[2], [dest], k)
        plsc.store_scatter(h[3], [dest], v)
        plsc.addupdate_scatter(h[5], [d], occ, mask=lastm)
    return 0

  lax.fori_loop(0, ntrip2, perm_body, 0)


def _body(scores_hbm, idx_hbm, maskw_hbm,
          rin_a0, rin_a1, rin_b0, rin_b1,
          key0, key1, hist0, hist1, off0, off1, exb0, exb1,
          cka0, cva0, ckb0, cvb0, cka1, cva1, ckb1, cvb1,
          cvout_a0, cvout_a1, cvout_b0, cvout_b1,
          maskw_a0, maskw_a1, maskw_b0, maskw_b1, sc0,
          sem_in_a, sem_in_b, sem_out_a, sem_out_b):
  wid = lax.axis_index("s") * 2 + lax.axis_index("c")
  lanes = _lanes()
  ones = jnp.ones((L,), jnp.int32)

  def one_pair(it, rin0, rin1, sem_in, rin_n0, rin_n1, sem_in_next,
               cvout0, cvout1, maskw0, maskw1, sem_out):
    q = it * NW + wid
    row1 = q + SK
    nv = (q + L) >> 4
    ntrip = (nv + 3) >> 2

    with jax.named_scope("sc_wait_in"):
      pltpu.make_async_copy(scores_hbm.at[q, pl.ds(0, SK)],
                            rin0.at[pl.ds(0, SK)], sem_in).wait()
      pltpu.make_async_copy(scores_hbm.at[row1, pl.ds(0, SK)],
                            rin1.at[pl.ds(0, SK)], sem_in).wait()

    @pl.when(it + 1 < NPAIR)
    def _():
      pltpu.async_copy(scores_hbm.at[q + NW, pl.ds(0, SK)],
                       rin_n0.at[pl.ds(0, SK)], sem_in_next)
      pltpu.async_copy(scores_hbm.at[row1 + NW, pl.ds(0, SK)],
                       rin_n1.at[pl.ds(0, SK)], sem_in_next)

    @pl.when(it >= 2)
    def _():
      qp = q - 2 * NW
      pltpu.make_async_copy(cvout0.at[pl.ds(0, TOP_K)], idx_hbm.at[qp],
                            sem_out).wait()
      pltpu.make_async_copy(maskw0, maskw_hbm.at[qp], sem_out).wait()
      pltpu.make_async_copy(cvout1.at[pl.ds(0, TOP_K)], idx_hbm.at[qp + SK],
                            sem_out).wait()
      pltpu.make_async_copy(maskw1, maskw_hbm.at[qp + SK], sem_out).wait()

    sc_p1 = jax.named_scope("sc_p1"); sc_p1.__enter__()
    _zero2(hist0, hist1, 256)

    @plsc.parallel_loop(0, (nv + 1) >> 1, unroll=2)
    def _(i):
      for rbuf, key_v, hist_v in ((rin0, key0, hist0), (rin1, key1, hist1)):
        for u in range(2):
          iv = i * 2 + u
          s = rbuf[pl.ds(iv * L, L)]
          ks = jnp.where(s < 0, s ^ 0x7FFFFFFF, s)
          valid = (iv * L + lanes) <= q
          ks = jnp.where(valid, ks, MIN_I32)
          key_v[pl.ds(iv * L, L)] = ks
          d1 = lax.shift_right_logical(ks ^ MIN_I32, 24)
          plsc.addupdate_scatter(hist_v, [d1], ones, mask=valid)
    sc_p1.__exit__(None, None, None)
    sc_sel = jax.named_scope("sc_select"); sc_sel.__enter__()

    sc0[0] = MIN_I32
    sc0[1] = q + 1
    sc0[2] = MIN_I32
    sc0[3] = q + 1

    @pl.when(q >= TOP_K - 1)
    def _():
      c1a, n_gt1a, _ = _find_bucket(hist0, jnp.int32(TOP_K))
      c1b, n_gt1b, _ = _find_bucket(hist1, jnp.int32(TOP_K))
      _zero2(hist0, hist1, 256)

      @plsc.parallel_loop(0, (nv + 1) >> 1, unroll=2)
      def _(i):
        for key_v, hist_v, c1 in ((key0, hist0, c1a), (key1, hist1, c1b)):
          for u in range(2):
            iv = i * 2 + u
            ks = key_v[pl.ds(iv * L, L)]
            ku = ks ^ MIN_I32
            m2 = (lax.shift_right_logical(ku, 24) == c1) & (ks != MIN_I32)
            d2 = lax.shift_right_logical(ku, 16) & 255
            plsc.addupdate_scatter(hist_v, [d2], ones, mask=m2)
      c2a, _g, n_ge2a = _find_bucket(hist0, TOP_K - n_gt1a)
      c2b, _g, n_ge2b = _find_bucket(hist1, TOP_K - n_gt1b)
      sc0[0] = ((c1a << 24) | (c2a << 16)) ^ MIN_I32
      sc0[1] = n_gt1a + n_ge2a
      sc0[2] = ((c1b << 24) | (c2b << 16)) ^ MIN_I32
      sc0[3] = n_gt1b + n_ge2b

    sc_sel.__exit__(None, None, None)
    sc_cp = jax.named_scope("sc_compact"); sc_cp.__enter__()
    ts0 = jnp.maximum(sc0[0], MIN_I32 + 1)
    ts1 = jnp.maximum(sc0[2], MIN_I32 + 1)

    fill_hi = jnp.minimum(
        ((jnp.maximum(jnp.maximum(sc0[1], sc0[3]), TOP_K) + 2 * L - 1) >> 5)
        * 2, CAP // L)
    for cka, cva, slot in ((cka0, cva0, 1), (cka1, cva1, 3)):
      @plsc.parallel_loop(sc0[slot] >> 4, fill_hi, unroll=2)
      def _(i):
        cka[pl.ds(i * L, L)] = jnp.full((L,), KU_NEG1E9, jnp.int32)
        cva[pl.ds(i * L, L)] = i * L + lanes

    z16 = jnp.zeros((L,), jnp.int32)

    @plsc.parallel_loop(0, (nv + 1) >> 1, unroll=2, carry=(z16, z16))
    def _(i, offs):
      off0, off1 = offs
      for u in range(2):
        iv = i * 2 + u
        ks = key0[pl.ds(iv * L, L)]
        m = ks >= ts0
        pos = plsc.cumsum(ones, mask=m)
        dest = off0 + pos - 1
        dm = m & (dest < CAP)
        plsc.store_scatter(cka0, [dest], ks ^ MIN_I32, mask=dm)
        plsc.store_scatter(cva0, [dest], iv * L + lanes, mask=dm)
        off0 = off0 + plsc.all_reduce_population_count(m)
        ks = key1[pl.ds(iv * L, L)]
        m = ks >= ts1
        pos = plsc.cumsum(ones, mask=m)
        dest = off1 + pos - 1
        dm = m & (dest < CAP)
        plsc.store_scatter(cka1, [dest], ks ^ MIN_I32, mask=dm)
        plsc.store_scatter(cva1, [dest], iv * L + lanes, mask=dm)
        off1 = off1 + plsc.all_reduce_population_count(m)
      return off0, off1
    sc_cp.__exit__(None, None, None)

    sc_st = jax.named_scope("sc_sort"); sc_st.__enter__()
    ncmax = jnp.maximum(jnp.maximum(sc0[1], sc0[3]), TOP_K)
    ntrip2 = jnp.minimum((ncmax + 2 * L - 1) >> 5, CAP // (2 * L))
    s0 = (cka0, cva0, ckb0, cvb0, hist0, off0, exb0)
    r0 = (ckb0, cvb0, cka0, cva0, hist0, off0, exb0)
    s1 = (cka1, cva1, ckb1, cvb1, hist1, off1, exb1)
    r1 = (ckb1, cvb1, cka1, cva1, hist1, off1, exb1)
    fin0 = (ckb0, cvb0, cka0, cvout0, hist0, off0, exb0)
    fin1 = (ckb1, cvb1, cka1, cvout1, hist1, off1, exb1)
    _radix_pass2(s0, s1, 0, ntrip2)
    _radix_pass2(r0, r1, 8, ntrip2)
    _radix_pass2(s0, s1, 16, ntrip2)
    _radix_pass2(fin0, fin1, 24, ntrip2)
    sc_st.__exit__(None, None, None)
    sc_mk = jax.named_scope("sc_mask"); sc_mk.__enter__()

    _zero2(maskw0, maskw1, SK // 4)

    @plsc.parallel_loop(0, TOP_K // L, unroll=2)
    def _(i):
      for cvout, maskw_v in ((cvout0, maskw0), (cvout1, maskw1)):
        j = cvout[pl.ds(i * L, L)]
        w = lax.shift_right_logical(j, 2)
        val = jnp.left_shift(ones, (j & 3) << 3)
        for t in range(4):
          plsc.addupdate_scatter(maskw_v, [w], val, mask=(j & 3) == t)

    sc_mk.__exit__(None, None, None)
    pltpu.async_copy(cvout0.at[pl.ds(0, TOP_K)], idx_hbm.at[q], sem_out)
    pltpu.async_copy(maskw0, maskw_hbm.at[q], sem_out)
    pltpu.async_copy(cvout1.at[pl.ds(0, TOP_K)], idx_hbm.at[row1], sem_out)
    pltpu.async_copy(maskw1, maskw_hbm.at[row1], sem_out)

  pltpu.async_copy(scores_hbm.at[wid, pl.ds(0, SK)], rin_a0.at[pl.ds(0, SK)],
                   sem_in_a)
  pltpu.async_copy(scores_hbm.at[wid + SK, pl.ds(0, SK)],
                   rin_a1.at[pl.ds(0, SK)], sem_in_a)

  def pair_body(i2, _):
    one_pair(2 * i2, rin_a0, rin_a1, sem_in_a, rin_b0, rin_b1, sem_in_b,
             cvout_a0, cvout_a1, maskw_a0, maskw_a1, sem_out_a)
    one_pair(2 * i2 + 1, rin_b0, rin_b1, sem_in_b, rin_a0, rin_a1, sem_in_a,
             cvout_b0, cvout_b1, maskw_b0, maskw_b1, sem_out_b)
    return 0

  lax.fori_loop(0, NPAIR // 2, pair_body, 0)

  for p, (cv0, cv1, mk0, mk1, sem_out) in enumerate(
      ((cvout_a0, cvout_a1, maskw_a0, maskw_a1, sem_out_a),
       (cvout_b0, cvout_b1, maskw_b0, maskw_b1, sem_out_b))):
    qlast = (NPAIR - 2 + p) * NW + wid
    pltpu.make_async_copy(cv0.at[pl.ds(0, TOP_K)], idx_hbm.at[qlast],
                          sem_out).wait()
    pltpu.make_async_copy(mk0, maskw_hbm.at[qlast], sem_out).wait()
    pltpu.make_async_copy(cv1.at[pl.ds(0, TOP_K)], idx_hbm.at[qlast + SK],
                          sem_out).wait()
    pltpu.make_async_copy(mk1, maskw_hbm.at[qlast + SK], sem_out).wait()


@functools.partial(jax.jit, static_argnums=())
def _topk_sc(scores):
  mesh = plsc.VectorSubcoreMesh(core_axis_name="c", subcore_axis_name="s")
  vm = lambda n: pltpu.VMEM((n,), jnp.int32)
  f = pl.kernel(
      _body,
      out_type=(
          jax.ShapeDtypeStruct((NROWS, TOP_K), jnp.int32),
          jax.ShapeDtypeStruct((NROWS, SK // 4), jnp.int32),
      ),
      mesh=mesh,
      compiler_params=pltpu.CompilerParams(needs_layout_passes=False),
      scratch_types=[
          vm(SKP), vm(SKP), vm(SKP), vm(SKP),
          vm(SKP), vm(SKP),
          vm(256), vm(256), vm(256), vm(256), vm(L), vm(L),
          vm(CAP), vm(CAP), vm(CAP), vm(CAP),
          vm(CAP), vm(CAP), vm(CAP), vm(CAP),
          vm(CAP), vm(CAP), vm(CAP), vm(CAP),
          vm(SK // 4), vm(SK // 4), vm(SK // 4), vm(SK // 4),
          pltpu.SMEM((8,), jnp.int32),
          pltpu.SemaphoreType.DMA,
          pltpu.SemaphoreType.DMA,
          pltpu.SemaphoreType.DMA,
          pltpu.SemaphoreType.DMA,
      ],
  )
  return f(scores)


def kernel(index_scores):
  B, Sq, Sk = index_scores.shape
  scores = lax.bitcast_convert_type(index_scores, jnp.int32).reshape(B * Sq, Sk)
  idx_out, maskw = _topk_sc(scores)
  top_k_indices = idx_out.reshape(B, Sq, TOP_K)
  mask_bytes = lax.bitcast_convert_type(maskw, jnp.uint8)
  top_k_mask = mask_bytes.reshape(B, Sq, Sk) != 0
  sparsity = jnp.float32(1.0) - jnp.float32(TOP_K) / jnp.float32(Sk)
  return (top_k_mask, top_k_indices, sparsity)


# speedup vs baseline: 1.6202x; 1.0022x over previous
import functools

import numpy as np
import jax
import jax.numpy as jnp
from jax import lax
from jax.experimental import pallas as pl
from jax.experimental.pallas import tpu as pltpu
from jax.experimental.pallas import tpu_sc as plsc

TOP_K = 512
SK = 4096
SKP = SK + 64
NROWS = 8192
NW = 32
NPAIR = SK // NW
L = 16
CAP = 576
MIN_I32 = -(2**31)

_S_NEG1E9 = int(np.float32(-1e9).view(np.int32))
KS_NEG1E9 = _S_NEG1E9 ^ 0x7FFFFFFF
KU_NEG1E9 = KS_NEG1E9 ^ MIN_I32


def _lanes():
  return lax.broadcasted_iota(jnp.int32, (L,), 0)


def _scalar_at(vec, lane):
  lanes = _lanes()
  return jnp.sum(jnp.where(lanes == lane, vec, 0))


def _find_bucket(hist_ref, target):
  lanes = _lanes()
  tv = plsc.load_gather(hist_ref, [lanes * L])
  for c in range(1, L):
    tv = tv + plsc.load_gather(hist_ref, [lanes * L + c])
  above = lax.rev(plsc.cumsum(lax.rev(tv, (0,))), (0,))
  excl = above - tv
  mstar = jnp.max(jnp.where(above >= target, lanes, -1))
  exm = _scalar_at(excl, mstar)
  h = hist_ref[pl.ds(mstar << 4, L)]
  rc = lax.rev(plsc.cumsum(lax.rev(h, (0,))), (0,))
  cum_ge = exm + rc
  c_in = jnp.max(jnp.where(cum_ge >= target, lanes, -1))
  c = (mstar << 4) | c_in
  n_ge = _scalar_at(cum_ge, c_in)
  hc = _scalar_at(h, c_in)
  return c, n_ge - hc, n_ge


def _zero2(ref0, ref1, nwords):
  z = jnp.zeros((L,), jnp.int32)

  @plsc.parallel_loop(0, nwords // (2 * L), unroll=2)
  def _(i):
    for u in range(2):
      ref0[pl.ds((i * 2 + u) * L, L)] = z
      ref1[pl.ds((i * 2 + u) * L, L)] = z


def _radix_pass2(b0, b1, shift, ntrip2):
  lanes = _lanes()
  ones = jnp.ones((L,), jnp.int32)
  _zero2(b0[4], b1[4], 256)

  @plsc.parallel_loop(0, ntrip2, unroll=2)
  def _(i):
    for h in (b0, b1):
      for u in range(2):
        k = h[0][pl.ds((i * 2 + u) * L, L)]
        d = lax.shift_right_logical(k, shift) & 255
        plsc.addupdate_scatter(h[4], [d], ones)

  for h in (b0, b1):
    tv = plsc.load_gather(h[4], [lanes * L])
    for c in range(1, L):
      tv = tv + plsc.load_gather(h[4], [lanes * L + c])
    above = lax.rev(plsc.cumsum(lax.rev(tv, (0,))), (0,))
    h[6][pl.ds(0, L)] = above - tv

  @plsc.parallel_loop(0, 16, unroll=2)
  def _(m):
    mi = jnp.full((L,), m, jnp.int32)
    for h in (b0, b1):
      hh = h[4][pl.ds(m * L, L)]
      rc = lax.rev(plsc.cumsum(lax.rev(hh, (0,))), (0,))
      carry = plsc.load_gather(h[6], [mi])
      h[5][pl.ds(m * L, L)] = carry + rc - hh

  def perm_body(i, _):
    for h in (b0, b1):
      for u in range(2):
        k = h[0][pl.ds((i * 2 + u) * L, L)]
        v = h[1][pl.ds((i * 2 + u) * L, L)]
        d = lax.shift_right_logical(k, shift) & 255
        occ, lastm = plsc.scan_count(d)
        base = plsc.load_gather(h[5], [d])
        dest = base + occ - 1
        plsc.store_scatter(h[2], [dest], k)
        plsc.store_scatter(h[3], [dest], v)
        plsc.addupdate_scatter(h[5], [d], occ, mask=lastm)
    return 0

  lax.fori_loop(0, ntrip2, perm_body, 0)


def _body(scores_hbm, idx_hbm, maskw_hbm,
          rin_a0, rin_a1, rin_b0, rin_b1,
          key0, key1, hist0, hist1, off0, off1, exb0, exb1,
          cka0, cva0, ckb0, cvb0, cka1, cva1, ckb1, cvb1,
          cvout_a0, cvout_a1, cvout_b0, cvout_b1,
          maskw_a0, maskw_a1, maskw_b0, maskw_b1, sc0,
          sem_in_a, sem_in_b, sem_out_a, sem_out_b):
  wid = lax.axis_index("s") * 2 + lax.axis_index("c")
  lanes = _lanes()
  ones = jnp.ones((L,), jnp.int32)

  def one_pair(it, rin0, rin1, sem_in, rin_n0, rin_n1, sem_in_next,
               cvout0, cvout1, maskw0, maskw1, sem_out):
    q = it * NW + wid
    row1 = q + SK
    nv = (q + L) >> 4

    pltpu.make_async_copy(scores_hbm.at[q, pl.ds(0, SK)],
                          rin0.at[pl.ds(0, SK)], sem_in).wait()
    pltpu.make_async_copy(scores_hbm.at[row1, pl.ds(0, SK)],
                          rin1.at[pl.ds(0, SK)], sem_in).wait()

    @pl.when(it + 1 < NPAIR)
    def _():
      pltpu.async_copy(scores_hbm.at[q + NW, pl.ds(0, SK)],
                       rin_n0.at[pl.ds(0, SK)], sem_in_next)
      pltpu.async_copy(scores_hbm.at[row1 + NW, pl.ds(0, SK)],
                       rin_n1.at[pl.ds(0, SK)], sem_in_next)

    @pl.when(it >= 2)
    def _():
      qp = q - 2 * NW
      pltpu.make_async_copy(cvout0.at[pl.ds(0, TOP_K)], idx_hbm.at[qp],
                            sem_out).wait()
      pltpu.make_async_copy(maskw0, maskw_hbm.at[qp], sem_out).wait()
      pltpu.make_async_copy(cvout1.at[pl.ds(0, TOP_K)], idx_hbm.at[qp + SK],
                            sem_out).wait()
      pltpu.make_async_copy(maskw1, maskw_hbm.at[qp + SK], sem_out).wait()

    _zero2(hist0, hist1, 256)

    @plsc.parallel_loop(0, (nv + 1) >> 1, unroll=2)
    def _(i):
      for rbuf, key_v, hist_v in ((rin0, key0, hist0), (rin1, key1, hist1)):
        for u in range(2):
          iv = i * 2 + u
          s = rbuf[pl.ds(iv * L, L)]
          ks = jnp.where(s < 0, s ^ 0x7FFFFFFF, s)
          valid = (iv * L + lanes) <= q
          ks = jnp.where(valid, ks, MIN_I32)
          key_v[pl.ds(iv * L, L)] = ks
          d1 = lax.shift_right_logical(ks ^ MIN_I32, 24)
          plsc.addupdate_scatter(hist_v, [d1], ones, mask=valid)

    sc0[0] = MIN_I32
    sc0[1] = q + 1
    sc0[2] = MIN_I32
    sc0[3] = q + 1

    @pl.when(q >= TOP_K - 1)
    def _():
      c1a, n_gt1a, _ = _find_bucket(hist0, jnp.int32(TOP_K))
      c1b, n_gt1b, _ = _find_bucket(hist1, jnp.int32(TOP_K))
      _zero2(hist0, hist1, 256)

      @plsc.parallel_loop(0, (nv + 1) >> 1, unroll=2)
      def _(i):
        for key_v, hist_v, c1 in ((key0, hist0, c1a), (key1, hist1, c1b)):
          for u in range(2):
            iv = i * 2 + u
            ks = key_v[pl.ds(iv * L, L)]
            ku = ks ^ MIN_I32
            m2 = (lax.shift_right_logical(ku, 24) == c1) & (ks != MIN_I32)
            d2 = lax.shift_right_logical(ku, 16) & 255
            plsc.addupdate_scatter(hist_v, [d2], ones, mask=m2)
      c2a, _g, n_ge2a = _find_bucket(hist0, TOP_K - n_gt1a)
      c2b, _g, n_ge2b = _find_bucket(hist1, TOP_K - n_gt1b)
      sc0[0] = ((c1a << 24) | (c2a << 16)) ^ MIN_I32
      sc0[1] = n_gt1a + n_ge2a
      sc0[2] = ((c1b << 24) | (c2b << 16)) ^ MIN_I32
      sc0[3] = n_gt1b + n_ge2b

    ts0 = jnp.maximum(sc0[0], MIN_I32 + 1)
    ts1 = jnp.maximum(sc0[2], MIN_I32 + 1)

    fill_hi = jnp.minimum(
        ((jnp.maximum(jnp.maximum(sc0[1], sc0[3]), TOP_K) + 2 * L - 1) >> 5)
        * 2, CAP // L)
    for cka, cva, slot in ((cka0, cva0, 1), (cka1, cva1, 3)):
      @plsc.parallel_loop(sc0[slot] >> 4, fill_hi, unroll=2)
      def _(i):
        cka[pl.ds(i * L, L)] = jnp.full((L,), KU_NEG1E9, jnp.int32)
        cva[pl.ds(i * L, L)] = i * L + lanes

    z16 = jnp.zeros((L,), jnp.int32)

    @plsc.parallel_loop(0, (nv + 1) >> 1, unroll=2, carry=(z16, z16))
    def _(i, offs):
      off0, off1 = offs
      for u in range(2):
        iv = i * 2 + u
        ks = key0[pl.ds(iv * L, L)]
        m = ks >= ts0
        pos = plsc.cumsum(ones, mask=m)
        dest = off0 + pos - 1
        dm = m & (dest < CAP)
        plsc.store_scatter(cka0, [dest], ks ^ MIN_I32, mask=dm)
        plsc.store_scatter(cva0, [dest], iv * L + lanes, mask=dm)
        off0 = off0 + plsc.all_reduce_population_count(m)
        ks = key1[pl.ds(iv * L, L)]
        m = ks >= ts1
        pos = plsc.cumsum(ones, mask=m)
        dest = off1 + pos - 1
        dm = m & (dest < CAP)
        plsc.store_scatter(cka1, [dest], ks ^ MIN_I32, mask=dm)
        plsc.store_scatter(cva1, [dest], iv * L + lanes, mask=dm)
        off1 = off1 + plsc.all_reduce_population_count(m)
      return off0, off1

    ncmax = jnp.maximum(jnp.maximum(sc0[1], sc0[3]), TOP_K)
    ntrip2 = jnp.minimum((ncmax + 2 * L - 1) >> 5, CAP // (2 * L))
    s0 = (cka0, cva0, ckb0, cvb0, hist0, off0, exb0)
    r0 = (ckb0, cvb0, cka0, cva0, hist0, off0, exb0)
    s1 = (cka1, cva1, ckb1, cvb1, hist1, off1, exb1)
    r1 = (ckb1, cvb1, cka1, cva1, hist1, off1, exb1)
    fin0 = (ckb0, cvb0, cka0, cvout0, hist0, off0, exb0)
    fin1 = (ckb1, cvb1, cka1, cvout1, hist1, off1, exb1)
    _radix_pass2(s0, s1, 0, ntrip2)
    _radix_pass2(r0, r1, 8, ntrip2)
    _radix_pass2(s0, s1, 16, ntrip2)
    _radix_pass2(fin0, fin1, 24, ntrip2)

    _zero2(maskw0, maskw1, SK // 4)

    @plsc.parallel_loop(0, TOP_K // L, unroll=2)
    def _(i):
      for cvout, maskw_v in ((cvout0, maskw0), (cvout1, maskw1)):
        j = cvout[pl.ds(i * L, L)]
        w = lax.shift_right_logical(j, 2)
        val = jnp.left_shift(ones, (j & 3) << 3)
        for t in range(4):
          plsc.addupdate_scatter(maskw_v, [w], val, mask=(j & 3) == t)

    pltpu.async_copy(cvout0.at[pl.ds(0, TOP_K)], idx_hbm.at[q], sem_out)
    pltpu.async_copy(maskw0, maskw_hbm.at[q], sem_out)
    pltpu.async_copy(cvout1.at[pl.ds(0, TOP_K)], idx_hbm.at[row1], sem_out)
    pltpu.async_copy(maskw1, maskw_hbm.at[row1], sem_out)

  pltpu.async_copy(scores_hbm.at[wid, pl.ds(0, SK)], rin_a0.at[pl.ds(0, SK)],
                   sem_in_a)
  pltpu.async_copy(scores_hbm.at[wid + SK, pl.ds(0, SK)],
                   rin_a1.at[pl.ds(0, SK)], sem_in_a)

  def pair_body(i2, _):
    one_pair(2 * i2, rin_a0, rin_a1, sem_in_a, rin_b0, rin_b1, sem_in_b,
             cvout_a0, cvout_a1, maskw_a0, maskw_a1, sem_out_a)
    one_pair(2 * i2 + 1, rin_b0, rin_b1, sem_in_b, rin_a0, rin_a1, sem_in_a,
             cvout_b0, cvout_b1, maskw_b0, maskw_b1, sem_out_b)
    return 0

  lax.fori_loop(0, NPAIR // 2, pair_body, 0)

  for p, (cv0, cv1, mk0, mk1, sem_out) in enumerate(
      ((cvout_a0, cvout_a1, maskw_a0, maskw_a1, sem_out_a),
       (cvout_b0, cvout_b1, maskw_b0, maskw_b1, sem_out_b))):
    qlast = (NPAIR - 2 + p) * NW + wid
    pltpu.make_async_copy(cv0.at[pl.ds(0, TOP_K)], idx_hbm.at[qlast],
                          sem_out).wait()
    pltpu.make_async_copy(mk0, maskw_hbm.at[qlast], sem_out).wait()
    pltpu.make_async_copy(cv1.at[pl.ds(0, TOP_K)], idx_hbm.at[qlast + SK],
                          sem_out).wait()
    pltpu.make_async_copy(mk1, maskw_hbm.at[qlast + SK], sem_out).wait()


@functools.partial(jax.jit, static_argnums=())
def _topk_sc(scores):
  mesh = plsc.VectorSubcoreMesh(core_axis_name="c", subcore_axis_name="s")
  vm = lambda n: pltpu.VMEM((n,), jnp.int32)
  f = pl.kernel(
      _body,
      out_type=(
          jax.ShapeDtypeStruct((NROWS, TOP_K), jnp.int32),
          jax.ShapeDtypeStruct((NROWS, SK // 4), jnp.int32),
      ),
      mesh=mesh,
      compiler_params=pltpu.CompilerParams(needs_layout_passes=False),
      scratch_types=[
          vm(SKP), vm(SKP), vm(SKP), vm(SKP),
          vm(SKP), vm(SKP),
          vm(256), vm(256), vm(256), vm(256), vm(L), vm(L),
          vm(CAP), vm(CAP), vm(CAP), vm(CAP),
          vm(CAP), vm(CAP), vm(CAP), vm(CAP),
          vm(CAP), vm(CAP), vm(CAP), vm(CAP),
          vm(SK // 4), vm(SK // 4), vm(SK // 4), vm(SK // 4),
          pltpu.SMEM((8,), jnp.int32),
          pltpu.SemaphoreType.DMA,
          pltpu.SemaphoreType.DMA,
          pltpu.SemaphoreType.DMA,
          pltpu.SemaphoreType.DMA,
      ],
  )
  return f(scores)


def kernel(index_scores):
  B, Sq, Sk = index_scores.shape
  scores = lax.bitcast_convert_type(index_scores, jnp.int32).reshape(B * Sq, Sk)
  idx_out, maskw = _topk_sc(scores)
  top_k_indices = idx_out.reshape(B, Sq, TOP_K)
  mask_bytes = lax.bitcast_convert_type(maskw, jnp.uint8)
  top_k_mask = mask_bytes.reshape(B, Sq, Sk) != 0
  sparsity = jnp.float32(1.0) - jnp.float32(TOP_K) / jnp.float32(Sk)
  return (top_k_mask, top_k_indices, sparsity)
